# Initial kernel scaffold; baseline (speedup 1.0000x reference)
import jax
import jax.numpy as jnp
from jax.experimental import pallas as pl


def kernel(h, r_feat, edge_feat, invar_ligand_shape, topo_out, e_w, hk_W1, hk_b1, hk_g, hk_beta, hk_W2, hk_b2, hv_W1, hv_b1, hv_g, hv_beta, hv_W2, hv_b2, hq_W1, hq_b1, hq_g, hq_beta, hq_W2, hq_b2, no_W1, no_b1, no_g, no_beta, no_W2, no_b2, edge_index):
    raise NotImplementedError("write your pallas kernel here")


# speedup vs baseline: 27.7946x; 27.7946x over previous
import functools

import jax
import jax.numpy as jnp
import numpy as np
from jax import lax
from jax.experimental import pallas as pl
from jax.experimental.pallas import tpu as pltpu
from jax.experimental.pallas import tpu_sc as plsc

N = 10000
E = 320000
D = 128
H = 8
DH = 16
RF = 64
EF = 4
SD = 15

NC = 2
NS = 16
NW = NC * NS
PERW = E // NW
CH = 80
NCHUNK = PERW // CH

NB = 1000
EB = 4000
NPAD = 10240
NU = 640
NACC = NPAD + NU

_EPS = 1e-5
_SCALE = 1.0 / np.sqrt(DH)


def _ln_relu(y, g, beta):
    mu = jnp.mean(y, axis=-1, keepdims=True)
    var = jnp.mean((y - mu) * (y - mu), axis=-1, keepdims=True)
    return jnp.maximum((y - mu) * lax.rsqrt(var + _EPS) * g + beta, 0.0)


def _pack2(a, b):
    au = lax.bitcast_convert_type(a, jnp.uint32)
    bu = lax.bitcast_convert_type(b, jnp.uint32)
    w = ((au + 0x8000) >> 16) | ((bu + 0x8000) & jnp.uint32(0xFFFF0000))
    return lax.bitcast_convert_type(w, jnp.int32)


def _unpack_lo(w):
    return lax.bitcast_convert_type(w << 16, jnp.float32)


def _unpack_hi(w):
    return lax.bitcast_convert_type(w & jnp.int32(-65536), jnp.float32)


def _node_tables_body(x_ref, h_ref, wd_ref, ws_ref, qw1_ref, qb1_ref, qg_ref,
                      qbeta_ref, qw2_ref, qb2_ref, td_ref, ts_ref):
    x = x_ref[...]
    h = h_ref[...]
    pd = jnp.dot(x, wd_ref[...], preferred_element_type=jnp.float32)
    ps = jnp.dot(h, ws_ref[...], preferred_element_type=jnp.float32)
    y = jnp.dot(h, qw1_ref[...], preferred_element_type=jnp.float32) + qb1_ref[...]
    y = _ln_relu(y, qg_ref[...], qbeta_ref[...])
    q = jnp.dot(y, qw2_ref[...], preferred_element_type=jnp.float32) + qb2_ref[...]
    qpad = jnp.concatenate([q, jnp.zeros((NB, D), jnp.float32)], axis=1)
    td_ref[...] = _pack2(pd, qpad)
    ts_ref[...] = _pack2(ps[:, :D], ps[:, D:])


def _node_tables(x, h, wd, ws, qw1, qb1, qg, qbeta, qw2, qb2):
    full = lambda s: pl.BlockSpec(s, lambda i: (0,) * len(s))
    return pl.pallas_call(
        _node_tables_body,
        grid=(N // NB,),
        in_specs=[
            pl.BlockSpec((NB, 272), lambda i: (i, 0)),
            pl.BlockSpec((NB, D), lambda i: (i, 0)),
            full((272, 2 * D)), full((D, 2 * D)),
            full((D, D)), full((1, D)), full((1, D)), full((1, D)),
            full((D, D)), full((1, D)),
        ],
        out_specs=[pl.BlockSpec((NB, 2 * D), lambda i: (i, 0)),
                   pl.BlockSpec((NB, D), lambda i: (i, 0))],
        out_shape=[jax.ShapeDtypeStruct((N, 2 * D), jnp.int32),
                   jax.ShapeDtypeStruct((N, D), jnp.int32)],
    )(x, h, wd, ws, qw1, qb1, qg, qbeta, qw2, qb2)


def _gather_body(td_hbm, ts_hbm, dstp_hbm, srcp_hbm, gd_hbm, gs_hbm,
                 idxd, idxs, rowsd, rowss, semd, sems):
    cid = lax.axis_index("c")
    sid = lax.axis_index("s")
    wid = sid * NC + cid
    base = wid * PERW
    pltpu.sync_copy(dstp_hbm.at[wid], idxd)
    pltpu.sync_copy(srcp_hbm.at[wid], idxs)
    pltpu.async_copy(td_hbm.at[idxd.at[0]], rowsd, semd)

    @pl.loop(0, NCHUNK)
    def _chunk(c):
        cn = lax.rem(c + 1, NCHUNK)
        pltpu.make_async_copy(td_hbm.at[idxd.at[c]], rowsd, semd).wait()
        pltpu.async_copy(ts_hbm.at[idxs.at[c]], rowss, sems)
        pltpu.sync_copy(rowsd, gd_hbm.at[pl.ds(base + c * CH, CH)])
        pltpu.make_async_copy(ts_hbm.at[idxs.at[c]], rowss, sems).wait()
        pltpu.async_copy(td_hbm.at[idxd.at[cn]], rowsd, semd)
        pltpu.sync_copy(rowss, gs_hbm.at[pl.ds(base + c * CH, CH)])

    pltpu.make_async_copy(td_hbm.at[idxd.at[0]], rowsd, semd).wait()


def _gather(td, ts, dstp, srcp):
    mesh = plsc.VectorSubcoreMesh(core_axis_name="c", subcore_axis_name="s",
                                  num_cores=NC, num_subcores=NS)
    f = pl.kernel(
        _gather_body,
        out_type=[jax.ShapeDtypeStruct((E, 2 * D), jnp.int32),
                  jax.ShapeDtypeStruct((E, D), jnp.int32)],
        mesh=mesh,
        scratch_types=[
            pltpu.VMEM((NCHUNK, CH), jnp.int32),
            pltpu.VMEM((NCHUNK, CH), jnp.int32),
            pltpu.VMEM((CH, 2 * D), jnp.int32),
            pltpu.VMEM((CH, D), jnp.int32),
            pltpu.SemaphoreType.DMA,
            pltpu.SemaphoreType.DMA,
        ],
    )
    return f(td, ts, dstp, srcp)


def _edge_body(er_ref, gd_ref, gs_ref, ew_ref, m16_ref, wer_ref, b1_ref,
               g_ref, beta_ref, wk2_ref, bk2_ref, wv2_ref, bv2_ref, msum_ref,
               mb_ref, mr_ref, mt_ref, zw_ref, zu_ref):
    gdi = gd_ref[...]
    gsi = gs_ref[...]
    y = (jnp.dot(er_ref[...], wer_ref[...], preferred_element_type=jnp.float32)
         + b1_ref[...] + _unpack_lo(gdi))
    g = g_ref[...]
    beta = beta_ref[...]
    ak = _ln_relu(y[:, :D] + _unpack_lo(gsi), g[:, :D], beta[:, :D])
    av = _ln_relu(y[:, D:] + _unpack_hi(gsi), g[:, D:], beta[:, D:])
    k = jnp.dot(ak, wk2_ref[...], preferred_element_type=jnp.float32) + bk2_ref[...]
    v = (jnp.dot(av, wv2_ref[...], preferred_element_type=jnp.float32)
         + bv2_ref[...]) * ew_ref[...]
    p = _unpack_hi(gdi[:, :D]) * k
    lg = jnp.dot(p, msum_ref[...], preferred_element_type=jnp.float32) * _SCALE
    u = jnp.exp(jnp.clip(lg, -70.0, 70.0))
    ub = jnp.dot(u, mb_ref[...], preferred_element_type=jnp.float32)
    zw_ref[...] = ub * v
    onehot = (m16_ref[...] == lax.broadcasted_iota(jnp.int32, (EB, NS), 1)
              .astype(jnp.float32))
    pexp = jnp.dot(onehot.astype(jnp.float32), mr_ref[...],
                   preferred_element_type=jnp.float32)
    uexp = jnp.dot(u, mt_ref[...], preferred_element_type=jnp.float32)
    zu_ref[...] = pexp * uexp


def _edge_stage(er, gd, gs, ew, m16, wer, b1, g, beta, wk2, bk2, wv2, bv2,
                msum, mb, mr, mt):
    full = lambda s: pl.BlockSpec(s, lambda i: (0,) * len(s))
    return pl.pallas_call(
        _edge_body,
        grid=(E // EB,),
        in_specs=[
            pl.BlockSpec((EB, EF + RF), lambda i: (i, 0)),
            pl.BlockSpec((EB, 2 * D), lambda i: (i, 0)),
            pl.BlockSpec((EB, D), lambda i: (i, 0)),
            pl.BlockSpec((EB, 1), lambda i: (i, 0)),
            pl.BlockSpec((EB, 1), lambda i: (i, 0)),
            full((EF + RF, 2 * D)), full((1, 2 * D)), full((1, 2 * D)),
            full((1, 2 * D)), full((D, D)), full((1, D)), full((D, D)),
            full((1, D)), full((D, H)), full((H, D)), full((NS, D)),
            full((H, D)),
        ],
        out_specs=[pl.BlockSpec((EB, D), lambda i: (i, 0)),
                   pl.BlockSpec((EB, D), lambda i: (i, 0))],
        out_shape=[jax.ShapeDtypeStruct((E, D), jnp.float32),
                   jax.ShapeDtypeStruct((E, D), jnp.float32)],
    )(er, gd, gs, ew, m16, wer, b1, g, beta, wk2, bk2, wv2, bv2, msum, mb,
      mr, mt)


SEG = 25
NSEG = NCHUNK // SEG
ZCH = 40
NZ = (NACC // NS) // ZCH


def _scatter_body(zw_hbm, zu_hbm, dstp_hbm, dup_hbm, zero_hbm, part_hbm,
                  ibufd, ibufu, zvw, zvu, semw, semu, acc):
    cid = lax.axis_index("c")
    sid = lax.axis_index("s")
    wid = sid * NC + cid
    base = wid * PERW
    rows = NACC // NS

    pltpu.sync_copy(zero_hbm, zvw.at[pl.ds(0, ZCH)])

    @pl.loop(0, NZ)
    def _z(i):
        pltpu.sync_copy(zvw.at[pl.ds(0, ZCH)],
                        acc.at[pl.ds(sid * rows + i * ZCH, ZCH)])

    plsc.subcore_barrier()

    @pl.loop(0, NSEG)
    def _seg(sg):
        pltpu.sync_copy(dstp_hbm.at[wid, sg], ibufd)
        pltpu.sync_copy(dup_hbm.at[wid, sg], ibufu)
        bsg = base + sg * SEG * CH
        pltpu.async_copy(zw_hbm.at[pl.ds(bsg, CH)], zvw, semw)

        @pl.loop(0, SEG)
        def _chunk(c):
            eb = bsg + c * CH
            en = bsg + lax.rem(c + 1, SEG) * CH
            pltpu.make_async_copy(zw_hbm.at[pl.ds(eb, CH)], zvw, semw).wait()
            pltpu.async_copy(zu_hbm.at[pl.ds(eb, CH)], zvu, semu)
            pltpu.sync_copy(zvw, acc.at[ibufd.at[c]], add=True)
            pltpu.make_async_copy(zu_hbm.at[pl.ds(eb, CH)], zvu, semu).wait()
            pltpu.async_copy(zw_hbm.at[pl.ds(en, CH)], zvw, semw)
            pltpu.sync_copy(zvu, acc.at[ibufu.at[c]], add=True)

        pltpu.make_async_copy(zw_hbm.at[pl.ds(bsg, CH)], zvw, semw).wait()

    plsc.subcore_barrier()

    @pl.loop(0, NZ)
    def _w(i):
        pltpu.sync_copy(acc.at[pl.ds(sid * rows + i * ZCH, ZCH)],
                        zvw.at[pl.ds(0, ZCH)])
        pltpu.sync_copy(zvw.at[pl.ds(0, ZCH)],
                        part_hbm.at[cid, pl.ds(sid * rows + i * ZCH, ZCH)])


def _scatter(zw, zu, dstp, dup, zero_blk):
    mesh = plsc.VectorSubcoreMesh(core_axis_name="c", subcore_axis_name="s",
                                  num_cores=NC, num_subcores=NS)
    f = pl.kernel(
        _scatter_body,
        out_type=jax.ShapeDtypeStruct((NC, NACC, D), jnp.float32),
        mesh=mesh,
        scratch_types=[
            pltpu.VMEM((SEG, CH), jnp.int32),
            pltpu.VMEM((SEG, CH), jnp.int32),
            pltpu.VMEM((CH, D), jnp.float32),
            pltpu.VMEM((CH, D), jnp.float32),
            pltpu.SemaphoreType.DMA,
            pltpu.SemaphoreType.DMA,
            pltpu.VMEM_SHARED((NACC, D), jnp.float32),
        ],
    )
    return f(zw, zu, dstp, dup, zero_blk)


def _final_body(p_ref, den_ref, h_ref, w1a_ref, w1h_ref, b1_ref, g_ref,
                beta_ref, w2_ref, b2_ref, mb_ref, out_ref):
    s = p_ref[0] + p_ref[1]
    den = den_ref[0] + den_ref[1]
    dinv = jnp.where(den > 0.0, 1.0 / den, 0.0)
    att = s * jnp.dot(dinv, mb_ref[...], preferred_element_type=jnp.float32)
    h = h_ref[...]
    y = (jnp.dot(att, w1a_ref[...], preferred_element_type=jnp.float32)
         + jnp.dot(h, w1h_ref[...], preferred_element_type=jnp.float32)
         + b1_ref[...])
    y = _ln_relu(y, g_ref[...], beta_ref[...])
    out_ref[...] = (jnp.dot(y, w2_ref[...], preferred_element_type=jnp.float32)
                    + b2_ref[...] + h)


def _final_stage(parts, den, h, w1a, w1h, b1, g, beta, w2, b2, mb):
    full = lambda s: pl.BlockSpec(s, lambda i: (0,) * len(s))
    return pl.pallas_call(
        _final_body,
        grid=(N // NB,),
        in_specs=[
            pl.BlockSpec((NC, NB, D), lambda i: (0, i, 0)),
            pl.BlockSpec((NC, NB, H), lambda i: (0, i, 0)),
            pl.BlockSpec((NB, D), lambda i: (i, 0)),
            full((D, D)), full((D, D)), full((1, D)), full((1, D)),
            full((1, D)), full((D, D)), full((1, D)), full((H, D)),
        ],
        out_specs=pl.BlockSpec((NB, D), lambda i: (i, 0)),
        out_shape=jax.ShapeDtypeStruct((N, D), jnp.float32),
    )(parts, den, h, w1a, w1h, b1, g, beta, w2, b2, mb)


def kernel(h, r_feat, edge_feat, invar_ligand_shape, topo_out, e_w,
           hk_W1, hk_b1, hk_g, hk_beta, hk_W2, hk_b2,
           hv_W1, hv_b1, hv_g, hv_beta, hv_W2, hv_b2,
           hq_W1, hq_b1, hq_g, hq_beta, hq_W2, hq_b2,
           no_W1, no_b1, no_g, no_beta, no_W2, no_b2,
           edge_index):
    f32 = jnp.float32
    src = edge_index[0].astype(jnp.int32)
    dst = edge_index[1].astype(jnp.int32)

    def splitw(w):
        o = EF + RF
        return w[:o], w[o:o + D], w[o + D:o + 2 * D], w[o + 2 * D:o + 3 * D], w[o + 3 * D:]

    ker, khd, khs, kt, ki = splitw(hk_W1)
    ver, vhd, vhs, vt, vi = splitw(hv_W1)
    wd = jnp.concatenate([jnp.concatenate([khd, kt, ki], 0),
                          jnp.concatenate([vhd, vt, vi], 0)], 1)
    wd = jnp.pad(wd, ((0, 1), (0, 0)))
    ws = jnp.concatenate([khs, vhs], 1)
    wer = jnp.concatenate([ker, ver], 1)
    b1kv = jnp.concatenate([hk_b1, hv_b1]).reshape(1, 2 * D)
    g_kv = jnp.concatenate([hk_g, hv_g]).reshape(1, 2 * D)
    beta_kv = jnp.concatenate([hk_beta, hv_beta]).reshape(1, 2 * D)

    x = jnp.concatenate([h, topo_out, invar_ligand_shape,
                         jnp.zeros((N, 1), f32)], axis=1)
    er = jnp.concatenate([edge_feat, r_feat], axis=1)

    lane = np.arange(D)
    msum = jnp.asarray((lane[:, None] // DH == np.arange(H)[None, :])
                       .astype(np.float32))
    mb = msum.T
    mr = jnp.asarray((lane[None, :] // H == np.arange(NS)[:, None])
                     .astype(np.float32))
    mt = jnp.asarray((lane[None, :] % H == np.arange(H)[:, None])
                     .astype(np.float32))

    dstp = dst.reshape(NW, NCHUNK, CH)
    srcp = src.reshape(NW, NCHUNK, CH)
    dstp4 = dst.reshape(NW, NSEG, SEG, CH)
    dup4 = (NPAD + dst // NS).reshape(NW, NSEG, SEG, CH)
    m16 = (dst % NS).astype(f32).reshape(E, 1)
    zero_blk = jnp.zeros((ZCH, D), f32)

    td, ts = _node_tables(x, h, wd, ws, hq_W1, hq_b1.reshape(1, D),
                          hq_g.reshape(1, D), hq_beta.reshape(1, D),
                          hq_W2, hq_b2.reshape(1, D))
    gd, gs = _gather(td, ts, dstp, srcp)
    zw, zu = _edge_stage(er, gd, gs, e_w.reshape(E, 1), m16, wer, b1kv, g_kv,
                         beta_kv, hk_W2, hk_b2.reshape(1, D), hv_W2,
                         hv_b2.reshape(1, D), msum, mb, mr, mt)
    parts = _scatter(zw, zu, dstp4, dup4, zero_blk)
    den = parts[:, NPAD:].reshape(NC, NU * NS, H)
    out = _final_stage(parts, den, h, no_W1[:D], no_W1[D:],
                       no_b1.reshape(1, D), no_g.reshape(1, D),
                       no_beta.reshape(1, D), no_W2, no_b2.reshape(1, D), mb)
    return out
